# baseline (speedup 1.0000x reference)
import functools

import jax
import jax.numpy as jnp
from jax import lax
from jax.experimental import pallas as pl
from jax.experimental.pallas import tpu as pltpu
from jax.experimental.pallas import tpu_sc as plsc

B = 16
V = 4096
NUM_GENES = 20000
LATENT = 64

NC = 2
NS = 16
L = 16
NW = NC * NS
GPT = NUM_GENES // NW
TBL = GPT * LATENT
CHUNK = 128
NVEC = V // L


def _sc_kernel(emb_hbm, gid_hbm, mask_hbm, out_hbm,
               ids_v, mask_v, cidx_v, clg_v, rows_v, table_v, sem):
    wid = lax.axis_index("s") * NC + lax.axis_index("c")
    lo = wid * GPT
    out_base = wid * TBL

    zeros16 = jnp.zeros((L,), jnp.float32)
    neginf16 = jnp.full((L,), -jnp.inf, jnp.float32)
    iota16 = lax.iota(jnp.int32, L)

    def _zf(i, _):
        table_v[pl.ds(i * L, L)] = zeros16
        return _
    lax.fori_loop(0, TBL // L, _zf, None)

    def _batch(b, _):
        pltpu.sync_copy(gid_hbm.at[b], ids_v)
        pltpu.sync_copy(mask_hbm.at[b], mask_v.at[pl.ds(0, V)])

        def _scan(i, cnt):
            ids = ids_v[pl.ds(i * L, L)]
            rel = ids - lo
            m = (rel >= 0) & (rel < GPT)
            gidx = b * V + i * L + iota16
            pos = cnt + plsc.cumsum(m.astype(jnp.int32)) - 1
            plsc.store_scatter(cidx_v, [pos], gidx, mask=m)
            plsc.store_scatter(clg_v, [pos], rel, mask=m)
            pc = plsc.all_reduce_population_count(m)
            return cnt + pc[0]
        cnt = lax.fori_loop(0, NVEC, _scan, 0)

        zi = jnp.zeros((L,), jnp.int32)
        for k in range(CHUNK // L):
            cidx_v[pl.ds(cnt + k * L, L)] = zi

        def _prime(j, _):
            tb = clg_v[pl.ds(j, L)][0] * LATENT
            for k in range(LATENT // L):
                table_v[pl.ds(tb + k * L, L)] = neginf16
            return _
        lax.fori_loop(0, cnt, _prime, None)

        def _chunk(c, _):
            cp = pltpu.make_async_copy(
                emb_hbm.at[cidx_v.at[pl.ds(c * CHUNK, CHUNK)]], rows_v, sem)
            cp.start()
            cp.wait()
            nrows = jnp.minimum(CHUNK, cnt - c * CHUNK)

            def _row(j, _):
                gi = cidx_v[pl.ds(c * CHUNK + j, L)][0]
                mval = mask_v[pl.ds(gi - b * V, L)][0]
                tb = clg_v[pl.ds(c * CHUNK + j, L)][0] * LATENT
                for k in range(LATENT // L):
                    row = rows_v[j, pl.ds(k * L, L)] * mval
                    cur = table_v[pl.ds(tb + k * L, L)]
                    table_v[pl.ds(tb + k * L, L)] = jnp.maximum(cur, row)
                return _
            lax.fori_loop(0, nrows, _row, None)
            return _
        nch = (cnt + CHUNK - 1) // CHUNK
        lax.fori_loop(0, nch, _chunk, None)

        pltpu.sync_copy(table_v, out_hbm.at[pl.ds(b * NUM_GENES * LATENT + out_base, TBL)])

        def _restore(j, _):
            tb = clg_v[pl.ds(j, L)][0] * LATENT
            for k in range(LATENT // L):
                table_v[pl.ds(tb + k * L, L)] = zeros16
            return _
        lax.fori_loop(0, cnt, _restore, None)
        return _

    lax.fori_loop(0, B, _batch, None)


@jax.jit
def kernel(variant_embeddings, gene_ids, mask):
    emb2d = variant_embeddings.reshape(B * V, LATENT)
    maskf = mask.astype(jnp.float32)
    mesh = plsc.VectorSubcoreMesh(
        core_axis_name="c", subcore_axis_name="s", num_cores=NC, num_subcores=NS)
    out = pl.kernel(
        _sc_kernel,
        out_type=jax.ShapeDtypeStruct((B * NUM_GENES * LATENT,), jnp.float32),
        mesh=mesh,
        compiler_params=pltpu.CompilerParams(needs_layout_passes=False, use_tc_tiling_on_sc=False),
        scratch_types=[
            pltpu.VMEM((V,), jnp.int32),
            pltpu.VMEM((V + L,), jnp.float32),
            pltpu.VMEM((V + CHUNK,), jnp.int32),
            pltpu.VMEM((V + CHUNK,), jnp.int32),
            pltpu.VMEM((CHUNK, LATENT), jnp.float32),
            pltpu.VMEM((TBL,), jnp.float32),
            pltpu.SemaphoreType.DMA,
        ],
    )(emb2d, gene_ids, maskf)
    return out.reshape(B, NUM_GENES, LATENT)


# speedup vs baseline: 1.0075x; 1.0075x over previous
import jax
import jax.numpy as jnp
from jax import lax
from jax.experimental import pallas as pl
from jax.experimental.pallas import tpu as pltpu
from jax.experimental.pallas import tpu_sc as plsc

B = 16
V = 4096
NUM_GENES = 20000
LATENT = 64

NC = 2
NS = 16
L = 16
NW = NC * NS
GPT = NUM_GENES // NW
TBL = GPT * LATENT
CHUNK = 128
NVEC = V // L
UNROLL = 4


def _sc_kernel(emb_hbm, gid_hbm, mask_hbm, out_hbm,
               ids_v, mask_v, cidx_v, clg_v, rows_v, table_v, gsem, osem0, osem1):
    wid = lax.axis_index("s") * NC + lax.axis_index("c")
    lo = wid * GPT
    out_base = wid * TBL

    zeros16 = jnp.zeros((L,), jnp.float32)
    neginf16 = jnp.full((L,), -jnp.inf, jnp.float32)
    iota16 = lax.iota(jnp.int32, L)
    ugpt = jnp.uint32(GPT)

    def _zf(i, _):
        for k in range(10):
            table_v[0, pl.ds((i * 10 + k) * L, L)] = zeros16
            table_v[1, pl.ds((i * 10 + k) * L, L)] = zeros16
        return _
    lax.fori_loop(0, TBL // L // 10, _zf, None)

    osems = (osem0, osem1)
    cnt_prev = [None, None]

    for b in range(B):
        p = b & 1
        cidx_p = cidx_v.at[p]
        clg_p = clg_v.at[p]
        table_p = table_v.at[p]

        pltpu.sync_copy(gid_hbm.at[b], ids_v)
        pltpu.sync_copy(mask_hbm.at[b], mask_v.at[pl.ds(0, V)])

        if b >= 2:
            pltpu.make_async_copy(
                table_p,
                out_hbm.at[pl.ds((b - 2) * NUM_GENES * LATENT + out_base, TBL)],
                osems[p]).wait()

            def _restore(j, _):
                tb = clg_p[pl.ds(j, L)][0] * LATENT
                for k in range(LATENT // L):
                    table_p[pl.ds(tb + k * L, L)] = zeros16
                return _
            lax.fori_loop(0, cnt_prev[p], _restore, None)

        base0 = b * V + iota16

        def _scan(i, cnt):
            for u in range(UNROLL):
                vi = i * UNROLL + u
                ids = ids_v[pl.ds(vi * L, L)]
                rel = ids - lo
                m = rel.astype(jnp.uint32) < ugpt
                pos = cnt + plsc.cumsum(m.astype(jnp.int32)) - 1
                plsc.store_scatter(cidx_p, [pos], base0 + vi * L, mask=m)
                plsc.store_scatter(clg_p, [pos], rel, mask=m)
                cnt = pos[15] + 1
            return cnt
        cnt = lax.fori_loop(0, NVEC // UNROLL, _scan, 0)

        zi = jnp.zeros((L,), jnp.int32)
        for k in range(CHUNK // L):
            cidx_p[pl.ds(cnt + k * L, L)] = zi

        nch = (cnt + CHUNK - 1) // CHUNK

        gcp0 = pltpu.make_async_copy(
            emb_hbm.at[cidx_p.at[pl.ds(0, CHUNK)]], rows_v, gsem)
        gcp0.start()

        def _prime(j, _):
            tb = clg_p[pl.ds(j, L)][0] * LATENT
            for k in range(LATENT // L):
                table_p[pl.ds(tb + k * L, L)] = neginf16
            return _
        lax.fori_loop(0, cnt, _prime, None)

        def _chunk(c, _):
            pltpu.make_async_copy(
                emb_hbm.at[cidx_p.at[pl.ds(c * CHUNK, CHUNK)]], rows_v, gsem).wait()
            nrows = jnp.minimum(CHUNK, cnt - c * CHUNK)

            def _row(j, _):
                gi = cidx_p[pl.ds(c * CHUNK + j, L)][0]
                mval = mask_v[pl.ds(gi - b * V, L)][0]
                tb = clg_p[pl.ds(c * CHUNK + j, L)][0] * LATENT
                for k in range(LATENT // L):
                    row = rows_v[j, pl.ds(k * L, L)] * mval
                    cur = table_p[pl.ds(tb + k * L, L)]
                    table_p[pl.ds(tb + k * L, L)] = jnp.maximum(cur, row)
                return _
            lax.fori_loop(0, nrows, _row, None)

            @pl.when(c + 1 < nch)
            def _fire_next():
                pltpu.make_async_copy(
                    emb_hbm.at[cidx_p.at[pl.ds((c + 1) * CHUNK, CHUNK)]],
                    rows_v, gsem).start()
            return _
        lax.fori_loop(0, nch, _chunk, None)

        pltpu.make_async_copy(
            table_p,
            out_hbm.at[pl.ds(b * NUM_GENES * LATENT + out_base, TBL)],
            osems[p]).start()
        cnt_prev[p] = cnt

    for b in (B - 2, B - 1):
        p = b & 1
        pltpu.make_async_copy(
            table_v.at[p],
            out_hbm.at[pl.ds(b * NUM_GENES * LATENT + out_base, TBL)],
            osems[p]).wait()


@jax.jit
def kernel(variant_embeddings, gene_ids, mask):
    emb2d = variant_embeddings.reshape(B * V, LATENT)
    maskf = mask.astype(jnp.float32)
    mesh = plsc.VectorSubcoreMesh(
        core_axis_name="c", subcore_axis_name="s", num_cores=NC, num_subcores=NS)
    out = pl.kernel(
        _sc_kernel,
        out_type=jax.ShapeDtypeStruct((B * NUM_GENES * LATENT,), jnp.float32),
        mesh=mesh,
        compiler_params=pltpu.CompilerParams(
            needs_layout_passes=False, use_tc_tiling_on_sc=False),
        scratch_types=[
            pltpu.VMEM((V,), jnp.int32),
            pltpu.VMEM((V + L,), jnp.float32),
            pltpu.VMEM((2, V + CHUNK), jnp.int32),
            pltpu.VMEM((2, V + CHUNK), jnp.int32),
            pltpu.VMEM((CHUNK, LATENT), jnp.float32),
            pltpu.VMEM((2, TBL), jnp.float32),
            pltpu.SemaphoreType.DMA,
            pltpu.SemaphoreType.DMA,
            pltpu.SemaphoreType.DMA,
        ],
    )(emb2d, gene_ids, maskf)
    return out.reshape(B, NUM_GENES, LATENT)


# speedup vs baseline: 1.0082x; 1.0007x over previous
import jax
import jax.numpy as jnp
from jax import lax
from jax.experimental import pallas as pl
from jax.experimental.pallas import tpu as pltpu
from jax.experimental.pallas import tpu_sc as plsc

B = 16
V = 4096
NUM_GENES = 20000
LATENT = 64

NC = 2
NS = 16
L = 16
NW = NC * NS
GPT = NUM_GENES // NW
TBL = GPT * LATENT
CHUNK = 128
NVEC = V // L
UNROLL = 4


def _sc_kernel(emb_hbm, gid_hbm, mask_hbm, out_hbm,
               ids_v, mask_v, cidx_v, clg_v, rows_v, table_v, gsem, osem0, osem1):
    wid = lax.axis_index("s") * NC + lax.axis_index("c")
    lo = wid * GPT
    out_base = wid * TBL

    zeros16 = jnp.zeros((L,), jnp.float32)
    neginf16 = jnp.full((L,), -jnp.inf, jnp.float32)
    iota16 = lax.iota(jnp.int32, L)
    ugpt = jnp.uint32(GPT)

    def _zf(i, _):
        for k in range(10):
            table_v[0, pl.ds((i * 10 + k) * L, L)] = zeros16
            table_v[1, pl.ds((i * 10 + k) * L, L)] = zeros16
        return _
    lax.fori_loop(0, TBL // L // 10, _zf, None)

    osems = (osem0, osem1)
    cnt_prev = [None, None]

    for b in range(B):
        p = b & 1
        cidx_p = cidx_v.at[p]
        clg_p = clg_v.at[p]
        table_p = table_v.at[p]

        with jax.named_scope("stage_ids"):
            pltpu.sync_copy(gid_hbm.at[b], ids_v)
            pltpu.sync_copy(mask_hbm.at[b], mask_v.at[pl.ds(0, V)])

        if b >= 2:
          with jax.named_scope("drain_restore"):
            pltpu.make_async_copy(
                table_p,
                out_hbm.at[pl.ds((b - 2) * NUM_GENES * LATENT + out_base, TBL)],
                osems[p]).wait()

            def _restore(j, _):
                tb = clg_p[pl.ds(j, L)][0] * LATENT
                for k in range(LATENT // L):
                    table_p[pl.ds(tb + k * L, L)] = zeros16
                return _
            lax.fori_loop(0, cnt_prev[p], _restore, None)

        base0 = b * V + iota16

        def _scan(i, cnt):
            for u in range(UNROLL):
                vi = i * UNROLL + u
                ids = ids_v[pl.ds(vi * L, L)]
                rel = ids - lo
                m = rel.astype(jnp.uint32) < ugpt
                pos = cnt + plsc.cumsum(m.astype(jnp.int32)) - 1
                plsc.store_scatter(cidx_p, [pos], base0 + vi * L, mask=m)
                plsc.store_scatter(clg_p, [pos], rel, mask=m)
                cnt = pos[15] + 1
            return cnt
        with jax.named_scope("scan"):
            cnt = lax.fori_loop(0, NVEC // UNROLL, _scan, 0)

        zi = jnp.zeros((L,), jnp.int32)
        for k in range(CHUNK // L):
            cidx_p[pl.ds(cnt + k * L, L)] = zi

        nch = (cnt + CHUNK - 1) // CHUNK

        gcp0 = pltpu.make_async_copy(
            emb_hbm.at[cidx_p.at[pl.ds(0, CHUNK)]], rows_v, gsem)
        gcp0.start()

        def _prime(j, _):
            tb = clg_p[pl.ds(j, L)][0] * LATENT
            for k in range(LATENT // L):
                table_p[pl.ds(tb + k * L, L)] = neginf16
            return _
        with jax.named_scope("prime"):
            lax.fori_loop(0, cnt, _prime, None)

        def _chunk(c, _):
            pltpu.make_async_copy(
                emb_hbm.at[cidx_p.at[pl.ds(c * CHUNK, CHUNK)]], rows_v, gsem).wait()
            nrows = jnp.minimum(CHUNK, cnt - c * CHUNK)

            def _row(j, _):
                gi = cidx_p[pl.ds(c * CHUNK + j, L)][0]
                mval = mask_v[pl.ds(gi - b * V, L)][0]
                tb = clg_p[pl.ds(c * CHUNK + j, L)][0] * LATENT
                for k in range(LATENT // L):
                    row = rows_v[j, pl.ds(k * L, L)] * mval
                    cur = table_p[pl.ds(tb + k * L, L)]
                    table_p[pl.ds(tb + k * L, L)] = jnp.maximum(cur, row)
                return _
            lax.fori_loop(0, nrows, _row, None)

            @pl.when(c + 1 < nch)
            def _fire_next():
                pltpu.make_async_copy(
                    emb_hbm.at[cidx_p.at[pl.ds((c + 1) * CHUNK, CHUNK)]],
                    rows_v, gsem).start()
            return _
        with jax.named_scope("chunks"):
            lax.fori_loop(0, nch, _chunk, None)

        pltpu.make_async_copy(
            table_p,
            out_hbm.at[pl.ds(b * NUM_GENES * LATENT + out_base, TBL)],
            osems[p]).start()
        cnt_prev[p] = cnt

    for b in (B - 2, B - 1):
        p = b & 1
        pltpu.make_async_copy(
            table_v.at[p],
            out_hbm.at[pl.ds(b * NUM_GENES * LATENT + out_base, TBL)],
            osems[p]).wait()


@jax.jit
def kernel(variant_embeddings, gene_ids, mask):
    emb2d = variant_embeddings.reshape(B * V, LATENT)
    maskf = mask.astype(jnp.float32)
    mesh = plsc.VectorSubcoreMesh(
        core_axis_name="c", subcore_axis_name="s", num_cores=NC, num_subcores=NS)
    out = pl.kernel(
        _sc_kernel,
        out_type=jax.ShapeDtypeStruct((B * NUM_GENES * LATENT,), jnp.float32),
        mesh=mesh,
        compiler_params=pltpu.CompilerParams(
            needs_layout_passes=False, use_tc_tiling_on_sc=False),
        scratch_types=[
            pltpu.VMEM((V,), jnp.int32),
            pltpu.VMEM((V + L,), jnp.float32),
            pltpu.VMEM((2, V + CHUNK), jnp.int32),
            pltpu.VMEM((2, V + CHUNK), jnp.int32),
            pltpu.VMEM((CHUNK, LATENT), jnp.float32),
            pltpu.VMEM((2, TBL), jnp.float32),
            pltpu.SemaphoreType.DMA,
            pltpu.SemaphoreType.DMA,
            pltpu.SemaphoreType.DMA,
        ],
    )(emb2d, gene_ids, maskf)
    return out.reshape(B, NUM_GENES, LATENT)
